# baseline (speedup 1.0000x reference)
import jax
import jax.numpy as jnp
from jax.experimental import pallas as pl
from jax.experimental.pallas import tpu as pltpu

_R = 8


def kernel(logits, indices):
    N, C, H, W = logits.shape
    D = H * W
    R = _R
    idx = indices.astype(jnp.int32)

    def body(idx_ref, *refs):
        x_refs = refs[:R]
        o_ref = refs[R]
        for k in range(R):
            plane = x_refs[k][...].reshape(1, D)
            o_ref[k] = jnp.broadcast_to(plane, (C, D))

    def in_map(k):
        return lambda i, idx_ref: (i * R + k, idx_ref[i * R + k], 0, 0)

    grid_spec = pltpu.PrefetchScalarGridSpec(
        num_scalar_prefetch=1,
        grid=(N // R,),
        in_specs=[pl.BlockSpec((1, 1, H, W), in_map(k)) for k in range(R)],
        out_specs=pl.BlockSpec((R, C, D), lambda i, idx_ref: (i, 0, 0)),
    )
    out = pl.pallas_call(
        body,
        grid_spec=grid_spec,
        out_shape=jax.ShapeDtypeStruct((N, C, D), logits.dtype),
    )(idx, *([logits] * R))
    return out.reshape(N, C, H, W)


# speedup vs baseline: 1.6583x; 1.6583x over previous
import jax
import jax.numpy as jnp
from jax.experimental import pallas as pl
from jax.experimental.pallas import tpu as pltpu

_R = 8


def kernel(logits, indices):
    N, C, H, W = logits.shape
    D = H * W
    R = _R
    x = logits.reshape(N, C, 1, D)
    idx = indices.astype(jnp.int32)

    def body(idx_ref, *refs):
        x_refs = refs[:R]
        o_ref = refs[R]
        for k in range(R):
            o_ref[k] = jnp.broadcast_to(x_refs[k][...].reshape(1, D), (C, D))

    def in_map(k):
        return lambda i, idx_ref: (i * R + k, idx_ref[i * R + k], 0, 0)

    grid_spec = pltpu.PrefetchScalarGridSpec(
        num_scalar_prefetch=1,
        grid=(N // R,),
        in_specs=[pl.BlockSpec((1, 1, 1, D), in_map(k)) for k in range(R)],
        out_specs=pl.BlockSpec((R, C, D), lambda i, idx_ref: (i, 0, 0)),
    )
    out = pl.pallas_call(
        body,
        grid_spec=grid_spec,
        out_shape=jax.ShapeDtypeStruct((N, C, D), logits.dtype),
        compiler_params=pltpu.CompilerParams(
            dimension_semantics=("parallel",),
        ),
    )(idx, *([x] * R))
    return out.reshape(N, C, H, W)


# speedup vs baseline: 1.7034x; 1.0272x over previous
import jax
import jax.numpy as jnp
from jax.experimental import pallas as pl
from jax.experimental.pallas import tpu as pltpu

_R = 16


def kernel(logits, indices):
    N, C, H, W = logits.shape
    D = H * W
    R = _R
    x = logits.reshape(N, C, 1, D)
    idx = indices.astype(jnp.int32)

    def body(idx_ref, *refs):
        x_refs = refs[:R]
        o_ref = refs[R]
        for k in range(R):
            o_ref[k] = jnp.broadcast_to(x_refs[k][...].reshape(1, D), (C, D))

    def in_map(k):
        return lambda i, idx_ref: (i * R + k, idx_ref[i * R + k], 0, 0)

    grid_spec = pltpu.PrefetchScalarGridSpec(
        num_scalar_prefetch=1,
        grid=(N // R,),
        in_specs=[pl.BlockSpec((1, 1, 1, D), in_map(k)) for k in range(R)],
        out_specs=pl.BlockSpec((R, C, D), lambda i, idx_ref: (i, 0, 0)),
    )
    out = pl.pallas_call(
        body,
        grid_spec=grid_spec,
        out_shape=jax.ShapeDtypeStruct((N, C, D), logits.dtype),
        compiler_params=pltpu.CompilerParams(
            dimension_semantics=("parallel",),
        ),
    )(idx, *([x] * R))
    return out.reshape(N, C, H, W)
